# baseline (speedup 1.0000x reference)
import functools

import jax
import jax.numpy as jnp
from jax import lax
from jax.experimental import pallas as pl
from jax.experimental.pallas import tpu as pltpu
from jax.experimental.pallas import tpu_sc as plsc


def _make_gather(num_entities, dim, batch):
    info = plsc.get_sparse_core_info()
    nc, ns = info.num_cores, info.num_subcores
    nw = nc * ns
    assert batch % nw == 0
    b_per_w = batch // nw
    chunk = 128 if b_per_w % 128 == 0 else b_per_w
    n_ch = b_per_w // chunk
    mesh = plsc.VectorSubcoreMesh(core_axis_name="c", subcore_axis_name="s")

    @functools.partial(
        pl.kernel,
        mesh=mesh,
        out_type=jax.ShapeDtypeStruct((batch, dim), jnp.float32),
        scratch_types=[
            pltpu.VMEM((n_ch, chunk), jnp.int32),
            pltpu.VMEM((b_per_w, dim), jnp.float32),
            pltpu.SemaphoreType.DMA,
            pltpu.SemaphoreType.DMA,
        ],
    )
    def gather_kernel(table_hbm, idx_hbm, out_hbm, idx_v, rows_v, sem_g, sem_o):
        wid = lax.axis_index("s") * nc + lax.axis_index("c")
        base = wid * b_per_w
        pltpu.sync_copy(idx_hbm.at[wid], idx_v)
        gathers = [
            pltpu.async_copy(
                table_hbm.at[idx_v.at[j]],
                rows_v.at[pl.ds(j * chunk, chunk)],
                sem_g,
            )
            for j in range(n_ch)
        ]
        writes = []
        for j in range(n_ch):
            gathers[j].wait()
            writes.append(
                pltpu.async_copy(
                    rows_v.at[pl.ds(j * chunk, chunk)],
                    out_hbm.at[pl.ds(base + j * chunk, chunk)],
                    sem_o,
                )
            )
        for w in writes:
            w.wait()

    return gather_kernel


_COPY_STRIPES = 16


def _copy_body(q_ref, o_ref, sem):
    batch = q_ref.shape[0]
    blk = batch // _COPY_STRIPES
    copies = [
        pltpu.make_async_copy(
            q_ref.at[pl.ds(i * blk, blk)], o_ref.at[pl.ds(i * blk, blk)], sem
        )
        for i in range(_COPY_STRIPES)
    ]
    for c in copies:
        c.start()
    for c in copies:
        c.wait()


def _tc_copy(question):
    batch, dim = question.shape
    return pl.pallas_call(
        _copy_body,
        in_specs=[pl.BlockSpec(memory_space=pl.ANY)],
        out_specs=pl.BlockSpec(memory_space=pl.ANY),
        scratch_shapes=[pltpu.SemaphoreType.DMA],
        out_shape=jax.ShapeDtypeStruct((batch, dim), question.dtype),
    )(question)


def kernel(entity_weight, question, head):
    num_entities, dim = entity_weight.shape
    batch = head.shape[0]
    gather = _make_gather(num_entities, dim, batch)
    info = plsc.get_sparse_core_info()
    nw = info.num_cores * info.num_subcores
    b_per_w = batch // nw
    chunk = 128 if b_per_w % 128 == 0 else b_per_w
    idx = head.astype(jnp.int32).reshape(nw, b_per_w // chunk, chunk)
    head_e = gather(entity_weight, idx)
    rel_e = _tc_copy(question)
    return (head_e, rel_e)


# speedup vs baseline: 6.8206x; 6.8206x over previous
import functools

import jax
import jax.numpy as jnp
from jax import lax
from jax.experimental import pallas as pl
from jax.experimental.pallas import tpu as pltpu
from jax.experimental.pallas import tpu_sc as plsc


def _make_gather(num_entities, dim, batch):
    info = plsc.get_sparse_core_info()
    nc, ns = info.num_cores, info.num_subcores
    nw = nc * ns
    assert batch % nw == 0
    b_per_w = batch // nw
    chunk = 128 if b_per_w % 128 == 0 else b_per_w
    n_ch = b_per_w // chunk
    mesh = plsc.VectorSubcoreMesh(core_axis_name="c", subcore_axis_name="s")

    @functools.partial(
        pl.kernel,
        mesh=mesh,
        out_type=jax.ShapeDtypeStruct((batch, dim), jnp.float32),
        scratch_types=[
            pltpu.VMEM((n_ch, chunk), jnp.int32),
            pltpu.VMEM((b_per_w, dim), jnp.float32),
            pltpu.SemaphoreType.DMA,
            pltpu.SemaphoreType.DMA,
        ],
    )
    def gather_kernel(table_hbm, idx_hbm, out_hbm, idx_v, rows_v, sem_g, sem_o):
        wid = lax.axis_index("s") * nc + lax.axis_index("c")
        base = wid * b_per_w
        pltpu.sync_copy(idx_hbm.at[wid], idx_v)
        gathers = [
            pltpu.async_copy(
                table_hbm.at[idx_v.at[j]],
                rows_v.at[pl.ds(j * chunk, chunk)],
                sem_g,
            )
            for j in range(n_ch)
        ]
        writes = []
        for j in range(n_ch):
            gathers[j].wait()
            writes.append(
                pltpu.async_copy(
                    rows_v.at[pl.ds(j * chunk, chunk)],
                    out_hbm.at[pl.ds(base + j * chunk, chunk)],
                    sem_o,
                )
            )
        for w in writes:
            w.wait()

    return gather_kernel


def _copy_body(q_ref, o_ref):
    o_ref[...] = q_ref[...]


def _tc_copy(question):
    batch, dim = question.shape
    grid = 32
    blk = batch // grid
    return pl.pallas_call(
        _copy_body,
        grid=(grid,),
        in_specs=[pl.BlockSpec((blk, dim), lambda i: (i, 0))],
        out_specs=pl.BlockSpec((blk, dim), lambda i: (i, 0)),
        out_shape=jax.ShapeDtypeStruct((batch, dim), question.dtype),
    )(question)


def kernel(entity_weight, question, head):
    num_entities, dim = entity_weight.shape
    batch = head.shape[0]
    gather = _make_gather(num_entities, dim, batch)
    info = plsc.get_sparse_core_info()
    nw = info.num_cores * info.num_subcores
    b_per_w = batch // nw
    chunk = 128 if b_per_w % 128 == 0 else b_per_w
    idx = head.astype(jnp.int32).reshape(nw, b_per_w // chunk, chunk)
    head_e = gather(entity_weight, idx)
    rel_e = _tc_copy(question)
    return (head_e, rel_e)


# speedup vs baseline: 9.0344x; 1.3246x over previous
import functools

import jax
import jax.numpy as jnp
from jax import lax
from jax.experimental import pallas as pl
from jax.experimental.pallas import tpu as pltpu
from jax.experimental.pallas import tpu_sc as plsc


def _make_gather(num_entities, dim, batch):
    info = plsc.get_sparse_core_info()
    nc, ns = info.num_cores, info.num_subcores
    nw = nc * ns
    assert batch % nw == 0
    b_per_w = batch // nw
    chunk = 128 if b_per_w % 128 == 0 else b_per_w
    n_ch = b_per_w // chunk
    mesh = plsc.VectorSubcoreMesh(core_axis_name="c", subcore_axis_name="s")

    @functools.partial(
        pl.kernel,
        mesh=mesh,
        out_type=jax.ShapeDtypeStruct((batch, dim), jnp.float32),
        scratch_types=[
            pltpu.VMEM((n_ch, chunk), jnp.int32),
            pltpu.VMEM((b_per_w, dim), jnp.float32),
            pltpu.SemaphoreType.DMA,
            pltpu.SemaphoreType.DMA,
        ],
    )
    def gather_kernel(table_hbm, idx_hbm, out_hbm, idx_v, rows_v, sem_g, sem_o):
        wid = lax.axis_index("s") * nc + lax.axis_index("c")
        base = wid * b_per_w
        pltpu.sync_copy(idx_hbm.at[wid], idx_v)
        gathers = [
            pltpu.async_copy(
                table_hbm.at[idx_v.at[j]],
                rows_v.at[pl.ds(j * chunk, chunk)],
                sem_g,
            )
            for j in range(n_ch)
        ]
        writes = []
        for j in range(n_ch):
            gathers[j].wait()
            writes.append(
                pltpu.async_copy(
                    rows_v.at[pl.ds(j * chunk, chunk)],
                    out_hbm.at[pl.ds(base + j * chunk, chunk)],
                    sem_o,
                )
            )
        for w in writes:
            w.wait()

    return gather_kernel


def _copy_body(q_ref, o_ref):
    o_ref[...] = q_ref[...]


def _tc_copy(question):
    batch, dim = question.shape
    grid = 4
    blk = batch // grid
    return pl.pallas_call(
        _copy_body,
        grid=(grid,),
        in_specs=[pl.BlockSpec((blk, dim), lambda i: (i, 0))],
        out_specs=pl.BlockSpec((blk, dim), lambda i: (i, 0)),
        out_shape=jax.ShapeDtypeStruct((batch, dim), question.dtype),
    )(question)


def kernel(entity_weight, question, head):
    num_entities, dim = entity_weight.shape
    batch = head.shape[0]
    gather = _make_gather(num_entities, dim, batch)
    info = plsc.get_sparse_core_info()
    nw = info.num_cores * info.num_subcores
    b_per_w = batch // nw
    chunk = 128 if b_per_w % 128 == 0 else b_per_w
    idx = head.astype(jnp.int32).reshape(nw, b_per_w // chunk, chunk)
    head_e = gather(entity_weight, idx)
    rel_e = _tc_copy(question)
    return (head_e, rel_e)


# speedup vs baseline: 9.2879x; 1.0281x over previous
import functools

import jax
import jax.numpy as jnp
from jax import lax
from jax.experimental import pallas as pl
from jax.experimental.pallas import tpu as pltpu
from jax.experimental.pallas import tpu_sc as plsc


def _make_gather(num_entities, dim, batch):
    info = plsc.get_sparse_core_info()
    nc, ns = info.num_cores, info.num_subcores
    nw = nc * ns
    assert batch % nw == 0
    b_per_w = batch // nw
    chunk = 128 if b_per_w % 128 == 0 else b_per_w
    n_ch = b_per_w // chunk
    mesh = plsc.VectorSubcoreMesh(core_axis_name="c", subcore_axis_name="s")

    @functools.partial(
        pl.kernel,
        mesh=mesh,
        out_type=jax.ShapeDtypeStruct((batch, dim), jnp.float32),
        scratch_types=[
            pltpu.VMEM((n_ch, chunk), jnp.int32),
            pltpu.VMEM((b_per_w, dim), jnp.float32),
            pltpu.SemaphoreType.DMA,
            pltpu.SemaphoreType.DMA,
        ],
    )
    def gather_kernel(table_hbm, idx_hbm, out_hbm, idx_v, rows_v, sem_g, sem_o):
        wid = lax.axis_index("s") * nc + lax.axis_index("c")
        base = wid * b_per_w
        pltpu.sync_copy(idx_hbm.at[wid], idx_v)
        gathers = [
            pltpu.async_copy(
                table_hbm.at[idx_v.at[j]],
                rows_v.at[pl.ds(j * chunk, chunk)],
                sem_g,
            )
            for j in range(n_ch)
        ]
        writes = []
        for j in range(n_ch):
            gathers[j].wait()
            writes.append(
                pltpu.async_copy(
                    rows_v.at[pl.ds(j * chunk, chunk)],
                    out_hbm.at[pl.ds(base + j * chunk, chunk)],
                    sem_o,
                )
            )
        for w in writes:
            w.wait()

    return gather_kernel


def _copy_body(q_ref, o_ref):
    o_ref[...] = q_ref[...]


def _tc_copy(question):
    batch, dim = question.shape
    grid = 2
    blk = batch // grid
    return pl.pallas_call(
        _copy_body,
        grid=(grid,),
        in_specs=[pl.BlockSpec((blk, dim), lambda i: (i, 0))],
        out_specs=pl.BlockSpec((blk, dim), lambda i: (i, 0)),
        out_shape=jax.ShapeDtypeStruct((batch, dim), question.dtype),
    )(question)


def kernel(entity_weight, question, head):
    num_entities, dim = entity_weight.shape
    batch = head.shape[0]
    gather = _make_gather(num_entities, dim, batch)
    info = plsc.get_sparse_core_info()
    nw = info.num_cores * info.num_subcores
    b_per_w = batch // nw
    chunk = 128 if b_per_w % 128 == 0 else b_per_w
    idx = head.astype(jnp.int32).reshape(nw, b_per_w // chunk, chunk)
    head_e = gather(entity_weight, idx)
    rel_e = _tc_copy(question)
    return (head_e, rel_e)
